# baseline (speedup 1.0000x reference)
import functools

import jax
import jax.numpy as jnp
from jax import lax
from jax.experimental import pallas as pl
from jax.experimental.pallas import tpu as pltpu
from jax.experimental.pallas import tpu_sc as plsc

NUM_EXPERTS = 64
MODEL_DIM = 768
EXPERT_DIM = 768
TOKENS = 4096

PAD8 = 8
TOKENS_PAD = TOKENS + NUM_EXPERTS * PAD8
BLK = 128
MAX_TILES = NUM_EXPERTS - 1 + TOKENS // BLK


_RBT = 512
_TBL = 128


def _router_body(x_ref, rw_ref, sel_ref, rank_ref, poff_ref,
                 e_ref, c_ref, lo_ref, hi_ref, carry_ref):
    t = pl.program_id(0)

    @pl.when(t == 0)
    def _():
        carry_ref[...] = jnp.zeros_like(carry_ref)

    logits = lax.dot_general(
        x_ref[...], rw_ref[...],
        dimension_numbers=(((1,), (1,)), ((), ())),
        preferred_element_type=jnp.float32,
    )
    m = jnp.max(logits, axis=1, keepdims=True)
    p = jnp.exp(logits - m)
    p = p / jnp.sum(p, axis=1, keepdims=True)
    pm = jnp.max(p, axis=1, keepdims=True)
    ids = lax.broadcasted_iota(jnp.int32, p.shape, 1)
    sel = jnp.min(jnp.where(p >= pm, ids, NUM_EXPERTS), axis=1, keepdims=True)

    oh = (ids == sel).astype(jnp.float32)
    ri = lax.broadcasted_iota(jnp.int32, (_RBT, _RBT), 0)
    rj = lax.broadcasted_iota(jnp.int32, (_RBT, _RBT), 1)
    ltri = (ri > rj).astype(jnp.float32)
    cum = lax.dot_general(ltri, oh, (((1,), (0,)), ((), ())),
                          preferred_element_type=jnp.float32)
    carry_f = carry_ref[...].astype(jnp.float32)
    rank = jnp.sum((cum + carry_f) * oh, axis=1, keepdims=True)

    sel_ref[...] = sel
    rank_ref[...] = rank.astype(jnp.int32)
    carry_ref[...] = carry_ref[...] + jnp.sum(
        oh, axis=0, keepdims=True).astype(jnp.int32)

    @pl.when(t == TOKENS // _RBT - 1)
    def _():
        counts = carry_ref[...]
        pc = ((counts + PAD8 - 1) // PAD8 * PAD8).astype(jnp.float32)
        ei = lax.broadcasted_iota(jnp.int32, (NUM_EXPERTS, NUM_EXPERTS), 0)
        ej = lax.broadcasted_iota(jnp.int32, (NUM_EXPERTS, NUM_EXPERTS), 1)
        excl = (ei < ej).astype(jnp.float32)
        incl = (ei <= ej).astype(jnp.float32)
        nn = (((1,), (0,)), ((), ()))
        poff = lax.dot_general(pc, excl, nn,
                               preferred_element_type=jnp.float32)
        tiles = ((counts + BLK - 1) // BLK).astype(jnp.float32)
        tcum = lax.dot_general(tiles, incl, nn,
                               preferred_element_type=jnp.float32)
        total = jnp.sum(tiles, axis=1, keepdims=True)
        poff_ref[...] = poff.astype(jnp.int32)

        tq = lax.broadcasted_iota(jnp.int32, (_TBL, 1), 0)
        tqf = tq.astype(jnp.float32)
        e_raw = jnp.sum((tqf >= tcum).astype(jnp.float32),
                        axis=1, keepdims=True).astype(jnp.int32)
        e_cl = jnp.minimum(e_raw, NUM_EXPERTS - 1)
        ohe = (lax.broadcasted_iota(jnp.int32, (_TBL, NUM_EXPERTS), 1)
               == e_cl).astype(jnp.float32)

        def pick(row):
            return jnp.sum(ohe * row, axis=1, keepdims=True)

        tcum_t = pick(tcum)
        tiles_t = pick(tiles)
        poff_t = pick(poff)
        cnt_t = pick(counts.astype(jnp.float32))
        i_t = tqf - (tcum_t - tiles_t)
        s_t = (poff_t + i_t * BLK).astype(jnp.int32)
        valid = jnp.clip((cnt_t - i_t * BLK).astype(jnp.int32), 0, BLK)
        c_t = jnp.minimum(s_t, TOKENS_PAD - BLK)
        lo = s_t - c_t
        hi = lo + valid
        is_pad = tqf >= total
        eids = lax.broadcasted_iota(jnp.int32, (1, NUM_EXPERTS), 1)
        e_last = jnp.max(jnp.where(counts > 0, eids, 0))
        e_ref[...] = jnp.where(is_pad, e_last, e_cl)
        c_ref[...] = jnp.where(is_pad, 0, c_t)
        lo_ref[...] = jnp.where(is_pad, 0, lo)
        hi_ref[...] = jnp.where(is_pad, 0, hi)


def _router(x, router_w):
    tbl_spec = pl.BlockSpec((_TBL, 1), lambda t: (0, 0))
    tbl_shape = jax.ShapeDtypeStruct((_TBL, 1), jnp.int32)
    return pl.pallas_call(
        _router_body,
        grid=(TOKENS // _RBT,),
        in_specs=[
            pl.BlockSpec((_RBT, MODEL_DIM), lambda t: (t, 0)),
            pl.BlockSpec((NUM_EXPERTS, MODEL_DIM), lambda t: (0, 0)),
        ],
        out_specs=[
            pl.BlockSpec((_RBT, 1), lambda t: (t, 0)),
            pl.BlockSpec((_RBT, 1), lambda t: (t, 0)),
            pl.BlockSpec((1, NUM_EXPERTS), lambda t: (0, 0)),
            tbl_spec, tbl_spec, tbl_spec, tbl_spec,
        ],
        out_shape=[
            jax.ShapeDtypeStruct((TOKENS, 1), jnp.int32),
            jax.ShapeDtypeStruct((TOKENS, 1), jnp.int32),
            jax.ShapeDtypeStruct((1, NUM_EXPERTS), jnp.int32),
            tbl_shape, tbl_shape, tbl_shape, tbl_shape,
        ],
        scratch_shapes=[pltpu.VMEM((1, NUM_EXPERTS), jnp.int32)],
        compiler_params=pltpu.CompilerParams(
            dimension_semantics=("arbitrary",)),
    )(x, router_w)


_SC_CORES = 2
_SC_SUBCORES = 16
_SC_WORKERS = _SC_CORES * _SC_SUBCORES
_SC_CHUNK = 128


def _sc_gather_rows(table, idx):
    n_rows = idx.shape[0]
    assert n_rows % (_SC_WORKERS * PAD8) == 0
    rows_pw = n_rows // _SC_WORKERS
    subs = [_SC_CHUNK] * (rows_pw // _SC_CHUNK)
    if rows_pw % _SC_CHUNK:
        subs.append(rows_pw % _SC_CHUNK)
    mesh = plsc.VectorSubcoreMesh(core_axis_name="c", subcore_axis_name="s")

    @functools.partial(
        pl.kernel,
        mesh=mesh,
        out_type=jax.ShapeDtypeStruct((n_rows, MODEL_DIM), jnp.float32),
        scratch_types=[
            *[pltpu.VMEM((s,), jnp.int32) for s in subs],
            pltpu.VMEM((rows_pw, MODEL_DIM), jnp.float32),
            pltpu.SemaphoreType.DMA,
        ],
    )
    def k(table_hbm, idx_hbm, out_hbm, *refs):
        idx_vs = refs[:len(subs)]
        rows_v, sem = refs[len(subs)], refs[len(subs) + 1]
        wid = lax.axis_index("s") * _SC_CORES + lax.axis_index("c")
        base = wid * rows_pw
        copies = []
        off = 0
        for s, idx_v in zip(subs, idx_vs):
            pltpu.sync_copy(idx_hbm.at[pl.ds(base + off, s)], idx_v)
            copies.append(
                pltpu.async_copy(table_hbm.at[idx_v],
                                 rows_v.at[pl.ds(off, s)], sem))
            off += s
        for c in copies:
            c.wait()
        pltpu.sync_copy(rows_v, out_hbm.at[pl.ds(base, rows_pw)])

    return k(table, idx)


def _sc_dispatch_scatter(x, sel, rank, poff):
    rows_pw = TOKENS // _SC_WORKERS
    mesh = plsc.VectorSubcoreMesh(core_axis_name="c", subcore_axis_name="s")

    @functools.partial(
        pl.kernel,
        mesh=mesh,
        out_type=[
            jax.ShapeDtypeStruct((TOKENS_PAD, MODEL_DIM), jnp.float32),
            jax.ShapeDtypeStruct((TOKENS,), jnp.int32),
        ],
        scratch_types=[
            pltpu.VMEM((rows_pw, MODEL_DIM), jnp.float32),
            pltpu.VMEM((rows_pw,), jnp.int32),
            pltpu.VMEM((rows_pw,), jnp.int32),
            pltpu.VMEM((rows_pw,), jnp.int32),
            pltpu.VMEM((rows_pw,), jnp.int32),
            pltpu.SemaphoreType.DMA,
            pltpu.SemaphoreType.DMA,
        ],
    )
    def k(x_hbm, sel_hbm, rank_hbm, poff_hbm, xs_hbm, slot_hbm,
          rows_v, e_v, r_v, pval_v, slot_v, sem, sem2):
        wid = lax.axis_index("s") * _SC_CORES + lax.axis_index("c")
        base = wid * rows_pw
        cp_rows = pltpu.async_copy(x_hbm.at[pl.ds(base, rows_pw)], rows_v, sem)
        cp_sel = pltpu.async_copy(sel_hbm.at[pl.ds(base, rows_pw)], e_v, sem2)
        cp_rank = pltpu.async_copy(rank_hbm.at[pl.ds(base, rows_pw)], r_v,
                                   sem2)
        cp_sel.wait()
        cp_rank.wait()
        pltpu.async_copy(poff_hbm.at[e_v], pval_v, sem2).wait()
        for i in range(rows_pw // 16):
            slot_v[pl.ds(i * 16, 16)] = (
                pval_v[pl.ds(i * 16, 16)] + r_v[pl.ds(i * 16, 16)])
        cp_slot = pltpu.async_copy(slot_v, slot_hbm.at[pl.ds(base, rows_pw)],
                                   sem2)
        cp_rows.wait()
        pltpu.async_copy(rows_v, xs_hbm.at[slot_v], sem).wait()
        cp_slot.wait()

    return k(x, sel, rank, poff)


def _gemm_body(e_ref, c_ref, lo_ref, hi_ref,
               xs_ref, w1_ref, w3_ref, w2_ref, out_ref):
    t = pl.program_id(0)
    c = pl.multiple_of(c_ref[t, 0], PAD8)
    lo = lo_ref[t, 0]
    hi = hi_ref[t, 0]

    @pl.when(hi > lo)
    def _():
        xb = xs_ref[pl.ds(c, BLK), :]
        w1e = w1_ref[0]
        w3e = w3_ref[0]
        w2e = w2_ref[0]
        nt = (((1,), (1,)), ((), ()))
        a = lax.dot_general(xb, w1e, nt, preferred_element_type=jnp.float32)
        b = lax.dot_general(xb, w3e, nt, preferred_element_type=jnp.float32)
        h = (a * jax.nn.sigmoid(a)) * b
        ob = lax.dot_general(h, w2e, nt, preferred_element_type=jnp.float32)
        rows = lax.broadcasted_iota(jnp.int32, (BLK, 1), 0)
        keep = (rows >= lo) & (rows < hi)
        cur = out_ref[pl.ds(c, BLK), :]
        out_ref[pl.ds(c, BLK), :] = jnp.where(keep, ob, cur)


def _grouped_gemm(e_arr, c_arr, lo_arr, hi_arr, xs, w1, w3, w2):
    spec = pltpu.PrefetchScalarGridSpec(
        num_scalar_prefetch=4,
        grid=(MAX_TILES,),
        in_specs=[
            pl.BlockSpec((TOKENS_PAD, MODEL_DIM), lambda t, e, c, l, h: (0, 0)),
            pl.BlockSpec((1, EXPERT_DIM, MODEL_DIM),
                         lambda t, e, c, l, h: (e[t, 0], 0, 0)),
            pl.BlockSpec((1, EXPERT_DIM, MODEL_DIM),
                         lambda t, e, c, l, h: (e[t, 0], 0, 0)),
            pl.BlockSpec((1, MODEL_DIM, EXPERT_DIM),
                         lambda t, e, c, l, h: (e[t, 0], 0, 0)),
        ],
        out_specs=pl.BlockSpec((TOKENS_PAD, MODEL_DIM),
                               lambda t, e, c, l, h: (0, 0)),
    )
    return pl.pallas_call(
        _gemm_body,
        grid_spec=spec,
        out_shape=jax.ShapeDtypeStruct((TOKENS_PAD, MODEL_DIM), jnp.float32),
        compiler_params=pltpu.CompilerParams(
            dimension_semantics=("arbitrary",)),
    )(e_arr, c_arr, lo_arr, hi_arr, xs, w1, w3, w2)


def kernel(x, router_w, w1, w2, w3):
    sel, rank, poff, e4, c4, lo4, hi4 = _router(x, router_w)

    xs, token_slot = _sc_dispatch_scatter(
        x, sel.reshape(TOKENS), rank.reshape(TOKENS),
        poff.reshape(NUM_EXPERTS))
    os_ = _grouped_gemm(e4, c4, lo4, hi4, xs, w1, w3, w2)
    return _sc_gather_rows(os_, token_slot)


# speedup vs baseline: 1.1218x; 1.1218x over previous
import functools

import jax
import jax.numpy as jnp
from jax import lax
from jax.experimental import pallas as pl
from jax.experimental.pallas import tpu as pltpu
from jax.experimental.pallas import tpu_sc as plsc

NUM_EXPERTS = 64
MODEL_DIM = 768
EXPERT_DIM = 768
TOKENS = 4096

PAD8 = 8
TOKENS_PAD = TOKENS + NUM_EXPERTS * PAD8
BLK = 128
MAX_TILES = NUM_EXPERTS - 1 + TOKENS // BLK


_RBT = 512
_TBL = 128


def _router_body(x_ref, rw_ref, slot_ref, e_ref, c_ref, lo_ref, hi_ref,
                 carry_ref, sel_all, rank_all):
    t = pl.program_id(0)

    @pl.when(t == 0)
    def _():
        carry_ref[...] = jnp.zeros_like(carry_ref)

    logits = lax.dot_general(
        x_ref[...], rw_ref[...],
        dimension_numbers=(((1,), (1,)), ((), ())),
        preferred_element_type=jnp.float32,
    )
    m = jnp.max(logits, axis=1, keepdims=True)
    p = jnp.exp(logits - m)
    p = p / jnp.sum(p, axis=1, keepdims=True)
    pm = jnp.max(p, axis=1, keepdims=True)
    ids = lax.broadcasted_iota(jnp.int32, p.shape, 1)
    sel = jnp.min(jnp.where(p >= pm, ids, NUM_EXPERTS), axis=1, keepdims=True)

    oh = (ids == sel).astype(jnp.float32)
    ri = lax.broadcasted_iota(jnp.int32, (_RBT, _RBT), 0)
    rj = lax.broadcasted_iota(jnp.int32, (_RBT, _RBT), 1)
    ltri = (ri > rj).astype(jnp.float32)
    cum = lax.dot_general(ltri, oh, (((1,), (0,)), ((), ())),
                          preferred_element_type=jnp.float32)
    carry_f = carry_ref[...].astype(jnp.float32)
    rank = jnp.sum((cum + carry_f) * oh, axis=1, keepdims=True)

    tb = pl.multiple_of(t * _RBT, _RBT)
    sel_all[pl.ds(tb, _RBT), :] = sel
    rank_all[pl.ds(tb, _RBT), :] = rank.astype(jnp.int32)
    carry_ref[...] = carry_ref[...] + jnp.sum(
        oh, axis=0, keepdims=True).astype(jnp.int32)

    @pl.when(t == TOKENS // _RBT - 1)
    def _():
        counts = carry_ref[...]
        pc = ((counts + PAD8 - 1) // PAD8 * PAD8).astype(jnp.float32)
        ei = lax.broadcasted_iota(jnp.int32, (NUM_EXPERTS, NUM_EXPERTS), 0)
        ej = lax.broadcasted_iota(jnp.int32, (NUM_EXPERTS, NUM_EXPERTS), 1)
        excl = (ei < ej).astype(jnp.float32)
        incl = (ei <= ej).astype(jnp.float32)
        nn = (((1,), (0,)), ((), ()))
        poff = lax.dot_general(pc, excl, nn,
                               preferred_element_type=jnp.float32)
        tiles = ((counts + BLK - 1) // BLK).astype(jnp.float32)
        tcum = lax.dot_general(tiles, incl, nn,
                               preferred_element_type=jnp.float32)
        total = jnp.sum(tiles, axis=1, keepdims=True)

        ids_all = lax.broadcasted_iota(jnp.int32, (TOKENS, NUM_EXPERTS), 1)
        oh_all = (ids_all == sel_all[...]).astype(jnp.float32)
        slot_ref[...] = (jnp.sum(oh_all * poff, axis=1, keepdims=True)
                         .astype(jnp.int32) + rank_all[...])

        tq = lax.broadcasted_iota(jnp.int32, (_TBL, 1), 0)
        tqf = tq.astype(jnp.float32)
        e_raw = jnp.sum((tqf >= tcum).astype(jnp.float32),
                        axis=1, keepdims=True).astype(jnp.int32)
        e_cl = jnp.minimum(e_raw, NUM_EXPERTS - 1)
        ohe = (lax.broadcasted_iota(jnp.int32, (_TBL, NUM_EXPERTS), 1)
               == e_cl).astype(jnp.float32)

        def pick(row):
            return jnp.sum(ohe * row, axis=1, keepdims=True)

        tcum_t = pick(tcum)
        tiles_t = pick(tiles)
        poff_t = pick(poff)
        cnt_t = pick(counts.astype(jnp.float32))
        i_t = tqf - (tcum_t - tiles_t)
        s_t = (poff_t + i_t * BLK).astype(jnp.int32)
        valid = jnp.clip((cnt_t - i_t * BLK).astype(jnp.int32), 0, BLK)
        c_t = jnp.minimum(s_t, TOKENS_PAD - BLK)
        lo = s_t - c_t
        hi = lo + valid
        is_pad = tqf >= total
        eids = lax.broadcasted_iota(jnp.int32, (1, NUM_EXPERTS), 1)
        e_last = jnp.max(jnp.where(counts > 0, eids, 0))
        e_ref[...] = jnp.where(is_pad, e_last, e_cl)
        c_ref[...] = jnp.where(is_pad, 0, c_t)
        lo_ref[...] = jnp.where(is_pad, 0, lo)
        hi_ref[...] = jnp.where(is_pad, 0, hi)


def _router(x, router_w):
    tbl_spec = pl.BlockSpec((_TBL, 1), lambda t: (0, 0))
    tbl_shape = jax.ShapeDtypeStruct((_TBL, 1), jnp.int32)
    return pl.pallas_call(
        _router_body,
        grid=(TOKENS // _RBT,),
        in_specs=[
            pl.BlockSpec((_RBT, MODEL_DIM), lambda t: (t, 0)),
            pl.BlockSpec((NUM_EXPERTS, MODEL_DIM), lambda t: (0, 0)),
        ],
        out_specs=[
            pl.BlockSpec((TOKENS, 1), lambda t: (0, 0)),
            tbl_spec, tbl_spec, tbl_spec, tbl_spec,
        ],
        out_shape=[
            jax.ShapeDtypeStruct((TOKENS, 1), jnp.int32),
            tbl_shape, tbl_shape, tbl_shape, tbl_shape,
        ],
        scratch_shapes=[
            pltpu.VMEM((1, NUM_EXPERTS), jnp.int32),
            pltpu.VMEM((TOKENS, 1), jnp.int32),
            pltpu.VMEM((TOKENS, 1), jnp.int32),
        ],
        compiler_params=pltpu.CompilerParams(
            dimension_semantics=("arbitrary",)),
    )(x, router_w)


_SC_CORES = 2
_SC_SUBCORES = 16
_SC_WORKERS = _SC_CORES * _SC_SUBCORES
_SC_CHUNK = 128


def _sc_gather_rows(table, idx):
    n_rows = idx.shape[0]
    assert n_rows % (_SC_WORKERS * PAD8) == 0
    rows_pw = n_rows // _SC_WORKERS
    subs = [_SC_CHUNK] * (rows_pw // _SC_CHUNK)
    if rows_pw % _SC_CHUNK:
        subs.append(rows_pw % _SC_CHUNK)
    mesh = plsc.VectorSubcoreMesh(core_axis_name="c", subcore_axis_name="s")

    @functools.partial(
        pl.kernel,
        mesh=mesh,
        out_type=jax.ShapeDtypeStruct((n_rows, MODEL_DIM), jnp.float32),
        scratch_types=[
            *[pltpu.VMEM((s,), jnp.int32) for s in subs],
            pltpu.VMEM((rows_pw, MODEL_DIM), jnp.float32),
            pltpu.SemaphoreType.DMA,
        ],
    )
    def k(table_hbm, idx_hbm, out_hbm, *refs):
        idx_vs = refs[:len(subs)]
        rows_v, sem = refs[len(subs)], refs[len(subs) + 1]
        wid = lax.axis_index("s") * _SC_CORES + lax.axis_index("c")
        base = wid * rows_pw
        copies = []
        off = 0
        for s, idx_v in zip(subs, idx_vs):
            pltpu.sync_copy(idx_hbm.at[pl.ds(base + off, s)], idx_v)
            copies.append(
                pltpu.async_copy(table_hbm.at[idx_v],
                                 rows_v.at[pl.ds(off, s)], sem))
            off += s
        for c in copies:
            c.wait()
        pltpu.sync_copy(rows_v, out_hbm.at[pl.ds(base, rows_pw)])

    return k(table, idx)


def _sc_scatter_rows(x, slot):
    rows_pw = TOKENS // _SC_WORKERS
    mesh = plsc.VectorSubcoreMesh(core_axis_name="c", subcore_axis_name="s")

    @functools.partial(
        pl.kernel,
        mesh=mesh,
        out_type=jax.ShapeDtypeStruct((TOKENS_PAD, MODEL_DIM), jnp.float32),
        scratch_types=[
            pltpu.VMEM((rows_pw, MODEL_DIM), jnp.float32),
            pltpu.VMEM((rows_pw,), jnp.int32),
            pltpu.SemaphoreType.DMA,
        ],
    )
    def k(x_hbm, slot_hbm, xs_hbm, rows_v, slot_v, sem):
        wid = lax.axis_index("s") * _SC_CORES + lax.axis_index("c")
        base = wid * rows_pw
        cp_rows = pltpu.async_copy(x_hbm.at[pl.ds(base, rows_pw)], rows_v, sem)
        pltpu.sync_copy(slot_hbm.at[pl.ds(base, rows_pw)], slot_v)
        cp_rows.wait()
        pltpu.async_copy(rows_v, xs_hbm.at[slot_v], sem).wait()

    return k(x, slot)


def _gemm_body(e_ref, c_ref, lo_ref, hi_ref,
               xs_ref, w1_ref, w3_ref, w2_ref, out_ref):
    t = pl.program_id(0)
    c = pl.multiple_of(c_ref[t], PAD8)
    lo = lo_ref[t]
    hi = hi_ref[t]

    @pl.when(hi > lo)
    def _():
        xb = xs_ref[pl.ds(c, BLK), :]
        w1e = w1_ref[0]
        w3e = w3_ref[0]
        w2e = w2_ref[0]
        nt = (((1,), (1,)), ((), ()))
        a = lax.dot_general(xb, w1e, nt, preferred_element_type=jnp.float32)
        b = lax.dot_general(xb, w3e, nt, preferred_element_type=jnp.float32)
        h = (a * jax.nn.sigmoid(a)) * b
        ob = lax.dot_general(h, w2e, nt, preferred_element_type=jnp.float32)
        rows = lax.broadcasted_iota(jnp.int32, (BLK, 1), 0)
        keep = (rows >= lo) & (rows < hi)
        cur = out_ref[pl.ds(c, BLK), :]
        out_ref[pl.ds(c, BLK), :] = jnp.where(keep, ob, cur)


def _grouped_gemm(e_arr, c_arr, lo_arr, hi_arr, xs, w1, w3, w2):
    spec = pltpu.PrefetchScalarGridSpec(
        num_scalar_prefetch=4,
        grid=(MAX_TILES,),
        in_specs=[
            pl.BlockSpec((TOKENS_PAD, MODEL_DIM), lambda t, e, c, l, h: (0, 0)),
            pl.BlockSpec((1, EXPERT_DIM, MODEL_DIM),
                         lambda t, e, c, l, h: (e[t], 0, 0)),
            pl.BlockSpec((1, EXPERT_DIM, MODEL_DIM),
                         lambda t, e, c, l, h: (e[t], 0, 0)),
            pl.BlockSpec((1, MODEL_DIM, EXPERT_DIM),
                         lambda t, e, c, l, h: (e[t], 0, 0)),
        ],
        out_specs=pl.BlockSpec((TOKENS_PAD, MODEL_DIM),
                               lambda t, e, c, l, h: (0, 0)),
    )
    return pl.pallas_call(
        _gemm_body,
        grid_spec=spec,
        out_shape=jax.ShapeDtypeStruct((TOKENS_PAD, MODEL_DIM), jnp.float32),
        compiler_params=pltpu.CompilerParams(
            dimension_semantics=("arbitrary",)),
    )(e_arr, c_arr, lo_arr, hi_arr, xs, w1, w3, w2)


def kernel(x, router_w, w1, w2, w3):
    slot4, e4, c4, lo4, hi4 = _router(x, router_w)
    slot = slot4.reshape(TOKENS)

    xs = _sc_scatter_rows(x, slot)
    os_ = _grouped_gemm(e4.reshape(_TBL), c4.reshape(_TBL),
                        lo4.reshape(_TBL), hi4.reshape(_TBL), xs, w1, w3, w2)
    return _sc_gather_rows(os_, slot)


# speedup vs baseline: 1.1267x; 1.0044x over previous
import functools

import jax
import jax.numpy as jnp
from jax import lax
from jax.experimental import pallas as pl
from jax.experimental.pallas import tpu as pltpu
from jax.experimental.pallas import tpu_sc as plsc

NUM_EXPERTS = 64
MODEL_DIM = 768
EXPERT_DIM = 768
TOKENS = 4096

PAD8 = 8
TOKENS_PAD = TOKENS + NUM_EXPERTS * PAD8
BLK = 128
MAX_TILES = NUM_EXPERTS - 1 + TOKENS // BLK


_RBT = 512
_TBL = 128


def _router_body(x_ref, rw_ref, slot_ref, tbl_ref,
                 carry_ref, sel_all, rank_all):
    t = pl.program_id(0)

    @pl.when(t == 0)
    def _():
        carry_ref[...] = jnp.zeros_like(carry_ref)

    logits = lax.dot_general(
        x_ref[...], rw_ref[...],
        dimension_numbers=(((1,), (1,)), ((), ())),
        preferred_element_type=jnp.float32,
    )
    m = jnp.max(logits, axis=1, keepdims=True)
    p = jnp.exp(logits - m)
    p = p / jnp.sum(p, axis=1, keepdims=True)
    pm = jnp.max(p, axis=1, keepdims=True)
    ids = lax.broadcasted_iota(jnp.int32, p.shape, 1)
    sel = jnp.min(jnp.where(p >= pm, ids, NUM_EXPERTS), axis=1, keepdims=True)

    oh = (ids == sel).astype(jnp.float32)
    ri = lax.broadcasted_iota(jnp.int32, (_RBT, _RBT), 0)
    rj = lax.broadcasted_iota(jnp.int32, (_RBT, _RBT), 1)
    ltri = (ri > rj).astype(jnp.float32)
    cum = lax.dot_general(ltri, oh, (((1,), (0,)), ((), ())),
                          preferred_element_type=jnp.float32)
    carry_f = carry_ref[...].astype(jnp.float32)
    rank = jnp.sum((cum + carry_f) * oh, axis=1, keepdims=True)

    tb = pl.multiple_of(t * _RBT, _RBT)
    sel_all[pl.ds(tb, _RBT), :] = sel
    rank_all[pl.ds(tb, _RBT), :] = rank.astype(jnp.int32)
    carry_ref[...] = carry_ref[...] + jnp.sum(
        oh, axis=0, keepdims=True).astype(jnp.int32)

    @pl.when(t == TOKENS // _RBT - 1)
    def _():
        counts = carry_ref[...]
        pc = ((counts + PAD8 - 1) // PAD8 * PAD8).astype(jnp.float32)
        ei = lax.broadcasted_iota(jnp.int32, (NUM_EXPERTS, NUM_EXPERTS), 0)
        ej = lax.broadcasted_iota(jnp.int32, (NUM_EXPERTS, NUM_EXPERTS), 1)
        excl = (ei < ej).astype(jnp.float32)
        incl = (ei <= ej).astype(jnp.float32)
        nn = (((1,), (0,)), ((), ()))
        poff = lax.dot_general(pc, excl, nn,
                               preferred_element_type=jnp.float32)
        tiles = ((counts + BLK - 1) // BLK).astype(jnp.float32)
        tcum = lax.dot_general(tiles, incl, nn,
                               preferred_element_type=jnp.float32)
        total = jnp.sum(tiles, axis=1, keepdims=True)

        ids_all = lax.broadcasted_iota(jnp.int32, (TOKENS, NUM_EXPERTS), 1)
        oh_all = (ids_all == sel_all[...]).astype(jnp.float32)
        slot_ref[...] = (jnp.sum(oh_all * poff, axis=1, keepdims=True)
                         .astype(jnp.int32) + rank_all[...])

        tq = lax.broadcasted_iota(jnp.int32, (_TBL, 1), 0)
        tqf = tq.astype(jnp.float32)
        e_raw = jnp.sum((tqf >= tcum).astype(jnp.float32),
                        axis=1, keepdims=True).astype(jnp.int32)
        e_cl = jnp.minimum(e_raw, NUM_EXPERTS - 1)
        ohe = (lax.broadcasted_iota(jnp.int32, (_TBL, NUM_EXPERTS), 1)
               == e_cl).astype(jnp.float32)

        def pick(row):
            return jnp.sum(ohe * row, axis=1, keepdims=True)

        tcum_t = pick(tcum)
        tiles_t = pick(tiles)
        poff_t = pick(poff)
        cnt_t = pick(counts.astype(jnp.float32))
        i_t = tqf - (tcum_t - tiles_t)
        s_t = (poff_t + i_t * BLK).astype(jnp.int32)
        valid = jnp.clip((cnt_t - i_t * BLK).astype(jnp.int32), 0, BLK)
        c_t = jnp.minimum(s_t, TOKENS_PAD - BLK)
        lo = s_t - c_t
        hi = lo + valid
        is_pad = tqf >= total
        eids = lax.broadcasted_iota(jnp.int32, (1, NUM_EXPERTS), 1)
        e_last = jnp.max(jnp.where(counts > 0, eids, 0))
        tbl_ref[pl.ds(0 * _TBL, _TBL), :] = jnp.where(is_pad, e_last, e_cl)
        tbl_ref[pl.ds(1 * _TBL, _TBL), :] = jnp.where(is_pad, 0, c_t)
        tbl_ref[pl.ds(2 * _TBL, _TBL), :] = jnp.where(is_pad, 0, lo)
        tbl_ref[pl.ds(3 * _TBL, _TBL), :] = jnp.where(is_pad, 0, hi)


def _router(x, router_w):
    return pl.pallas_call(
        _router_body,
        grid=(TOKENS // _RBT,),
        in_specs=[
            pl.BlockSpec((_RBT, MODEL_DIM), lambda t: (t, 0)),
            pl.BlockSpec((NUM_EXPERTS, MODEL_DIM), lambda t: (0, 0)),
        ],
        out_specs=[
            pl.BlockSpec((TOKENS, 1), lambda t: (0, 0)),
            pl.BlockSpec((4 * _TBL, 1), lambda t: (0, 0)),
        ],
        out_shape=[
            jax.ShapeDtypeStruct((TOKENS, 1), jnp.int32),
            jax.ShapeDtypeStruct((4 * _TBL, 1), jnp.int32),
        ],
        scratch_shapes=[
            pltpu.VMEM((1, NUM_EXPERTS), jnp.int32),
            pltpu.VMEM((TOKENS, 1), jnp.int32),
            pltpu.VMEM((TOKENS, 1), jnp.int32),
        ],
        compiler_params=pltpu.CompilerParams(
            dimension_semantics=("arbitrary",)),
    )(x, router_w)


_SC_CORES = 2
_SC_SUBCORES = 16
_SC_WORKERS = _SC_CORES * _SC_SUBCORES
_SC_CHUNK = 128


def _sc_gather_rows(table, idx):
    n_rows = idx.shape[0]
    assert n_rows % (_SC_WORKERS * PAD8) == 0
    rows_pw = n_rows // _SC_WORKERS
    subs = [_SC_CHUNK] * (rows_pw // _SC_CHUNK)
    if rows_pw % _SC_CHUNK:
        subs.append(rows_pw % _SC_CHUNK)
    mesh = plsc.VectorSubcoreMesh(core_axis_name="c", subcore_axis_name="s")

    @functools.partial(
        pl.kernel,
        mesh=mesh,
        out_type=jax.ShapeDtypeStruct((n_rows, MODEL_DIM), jnp.float32),
        scratch_types=[
            *[pltpu.VMEM((s,), jnp.int32) for s in subs],
            pltpu.VMEM((rows_pw, MODEL_DIM), jnp.float32),
            pltpu.SemaphoreType.DMA,
        ],
    )
    def k(table_hbm, idx_hbm, out_hbm, *refs):
        idx_vs = refs[:len(subs)]
        rows_v, sem = refs[len(subs)], refs[len(subs) + 1]
        wid = lax.axis_index("s") * _SC_CORES + lax.axis_index("c")
        base = wid * rows_pw
        copies = []
        off = 0
        for s, idx_v in zip(subs, idx_vs):
            pltpu.sync_copy(idx_hbm.at[pl.ds(base + off, s)], idx_v)
            copies.append(
                pltpu.async_copy(table_hbm.at[idx_v],
                                 rows_v.at[pl.ds(off, s)], sem))
            off += s
        for c in copies:
            c.wait()
        pltpu.sync_copy(rows_v, out_hbm.at[pl.ds(base, rows_pw)])

    return k(table, idx)


def _sc_scatter_rows(x, slot):
    rows_pw = TOKENS // _SC_WORKERS
    mesh = plsc.VectorSubcoreMesh(core_axis_name="c", subcore_axis_name="s")

    @functools.partial(
        pl.kernel,
        mesh=mesh,
        out_type=jax.ShapeDtypeStruct((TOKENS_PAD, MODEL_DIM), jnp.float32),
        scratch_types=[
            pltpu.VMEM((rows_pw, MODEL_DIM), jnp.float32),
            pltpu.VMEM((rows_pw,), jnp.int32),
            pltpu.SemaphoreType.DMA,
        ],
    )
    def k(x_hbm, slot_hbm, xs_hbm, rows_v, slot_v, sem):
        wid = lax.axis_index("s") * _SC_CORES + lax.axis_index("c")
        base = wid * rows_pw
        cp_rows = pltpu.async_copy(x_hbm.at[pl.ds(base, rows_pw)], rows_v, sem)
        pltpu.sync_copy(slot_hbm.at[pl.ds(base, rows_pw)], slot_v)
        cp_rows.wait()
        pltpu.async_copy(rows_v, xs_hbm.at[slot_v], sem).wait()

    return k(x, slot)


def _gemm_body(e_ref, c_ref, lo_ref, hi_ref,
               xs_ref, w1_ref, w3_ref, w2_ref, out_ref):
    t = pl.program_id(0)
    c = pl.multiple_of(c_ref[t], PAD8)
    lo = lo_ref[t]
    hi = hi_ref[t]

    @pl.when(hi > lo)
    def _():
        xb = xs_ref[pl.ds(c, BLK), :]
        w1e = w1_ref[0]
        w3e = w3_ref[0]
        w2e = w2_ref[0]
        nt = (((1,), (1,)), ((), ()))
        a = lax.dot_general(xb, w1e, nt, preferred_element_type=jnp.float32)
        b = lax.dot_general(xb, w3e, nt, preferred_element_type=jnp.float32)
        h = (a * jax.nn.sigmoid(a)) * b
        ob = lax.dot_general(h, w2e, nt, preferred_element_type=jnp.float32)
        rows = lax.broadcasted_iota(jnp.int32, (BLK, 1), 0)
        keep = (rows >= lo) & (rows < hi)
        cur = out_ref[pl.ds(c, BLK), :]
        out_ref[pl.ds(c, BLK), :] = jnp.where(keep, ob, cur)


def _grouped_gemm(e_arr, c_arr, lo_arr, hi_arr, xs, w1, w3, w2):
    spec = pltpu.PrefetchScalarGridSpec(
        num_scalar_prefetch=4,
        grid=(MAX_TILES,),
        in_specs=[
            pl.BlockSpec((TOKENS_PAD, MODEL_DIM), lambda t, e, c, l, h: (0, 0)),
            pl.BlockSpec((1, EXPERT_DIM, MODEL_DIM),
                         lambda t, e, c, l, h: (e[t], 0, 0)),
            pl.BlockSpec((1, EXPERT_DIM, MODEL_DIM),
                         lambda t, e, c, l, h: (e[t], 0, 0)),
            pl.BlockSpec((1, MODEL_DIM, EXPERT_DIM),
                         lambda t, e, c, l, h: (e[t], 0, 0)),
        ],
        out_specs=pl.BlockSpec((TOKENS_PAD, MODEL_DIM),
                               lambda t, e, c, l, h: (0, 0)),
    )
    return pl.pallas_call(
        _gemm_body,
        grid_spec=spec,
        out_shape=jax.ShapeDtypeStruct((TOKENS_PAD, MODEL_DIM), jnp.float32),
        compiler_params=pltpu.CompilerParams(
            dimension_semantics=("arbitrary",)),
    )(e_arr, c_arr, lo_arr, hi_arr, xs, w1, w3, w2)


def kernel(x, router_w, w1, w2, w3):
    slot4, tbl = _router(x, router_w)
    slot = slot4.reshape(TOKENS)
    tbl = tbl.reshape(4 * _TBL)

    xs = _sc_scatter_rows(x, slot)
    os_ = _grouped_gemm(tbl[:_TBL], tbl[_TBL:2 * _TBL],
                        tbl[2 * _TBL:3 * _TBL], tbl[3 * _TBL:],
                        xs, w1, w3, w2)
    return _sc_gather_rows(os_, slot)


# speedup vs baseline: 1.1320x; 1.0048x over previous
import functools

import jax
import jax.numpy as jnp
from jax import lax
from jax.experimental import pallas as pl
from jax.experimental.pallas import tpu as pltpu
from jax.experimental.pallas import tpu_sc as plsc

NUM_EXPERTS = 64
MODEL_DIM = 768
EXPERT_DIM = 768
TOKENS = 4096

PAD8 = 8
TOKENS_PAD = TOKENS + NUM_EXPERTS * PAD8
BLK = 128
MAX_TILES = NUM_EXPERTS - 1 + TOKENS // BLK


_RBT = 1024
_TBL = 128


def _router_body(x_ref, rw_ref, slot_ref, tbl_ref,
                 carry_ref, sel_all, rank_all):
    t = pl.program_id(0)

    @pl.when(t == 0)
    def _():
        carry_ref[...] = jnp.zeros_like(carry_ref)

    logits = lax.dot_general(
        x_ref[...], rw_ref[...],
        dimension_numbers=(((1,), (1,)), ((), ())),
        preferred_element_type=jnp.float32,
    )
    m = jnp.max(logits, axis=1, keepdims=True)
    p = jnp.exp(logits - m)
    p = p / jnp.sum(p, axis=1, keepdims=True)
    pm = jnp.max(p, axis=1, keepdims=True)
    ids = lax.broadcasted_iota(jnp.int32, p.shape, 1)
    sel = jnp.min(jnp.where(p >= pm, ids, NUM_EXPERTS), axis=1, keepdims=True)

    oh = (ids == sel).astype(jnp.float32)
    ri = lax.broadcasted_iota(jnp.int32, (_RBT, _RBT), 0)
    rj = lax.broadcasted_iota(jnp.int32, (_RBT, _RBT), 1)
    ltri = (ri > rj).astype(jnp.float32)
    cum = lax.dot_general(ltri, oh, (((1,), (0,)), ((), ())),
                          preferred_element_type=jnp.float32)
    carry_f = carry_ref[...].astype(jnp.float32)
    rank = jnp.sum((cum + carry_f) * oh, axis=1, keepdims=True)

    tb = pl.multiple_of(t * _RBT, _RBT)
    sel_all[pl.ds(tb, _RBT), :] = sel
    rank_all[pl.ds(tb, _RBT), :] = rank.astype(jnp.int32)
    carry_ref[...] = carry_ref[...] + jnp.sum(
        oh, axis=0, keepdims=True).astype(jnp.int32)

    @pl.when(t == TOKENS // _RBT - 1)
    def _():
        counts = carry_ref[...]
        pc = ((counts + PAD8 - 1) // PAD8 * PAD8).astype(jnp.float32)
        ei = lax.broadcasted_iota(jnp.int32, (NUM_EXPERTS, NUM_EXPERTS), 0)
        ej = lax.broadcasted_iota(jnp.int32, (NUM_EXPERTS, NUM_EXPERTS), 1)
        excl = (ei < ej).astype(jnp.float32)
        incl = (ei <= ej).astype(jnp.float32)
        nn = (((1,), (0,)), ((), ()))
        poff = lax.dot_general(pc, excl, nn,
                               preferred_element_type=jnp.float32)
        tiles = ((counts + BLK - 1) // BLK).astype(jnp.float32)
        tcum = lax.dot_general(tiles, incl, nn,
                               preferred_element_type=jnp.float32)
        total = jnp.sum(tiles, axis=1, keepdims=True)

        ids_all = lax.broadcasted_iota(jnp.int32, (TOKENS, NUM_EXPERTS), 1)
        oh_all = (ids_all == sel_all[...]).astype(jnp.float32)
        slot_ref[...] = (jnp.sum(oh_all * poff, axis=1, keepdims=True)
                         .astype(jnp.int32) + rank_all[...])

        tq = lax.broadcasted_iota(jnp.int32, (_TBL, 1), 0)
        tqf = tq.astype(jnp.float32)
        e_raw = jnp.sum((tqf >= tcum).astype(jnp.float32),
                        axis=1, keepdims=True).astype(jnp.int32)
        e_cl = jnp.minimum(e_raw, NUM_EXPERTS - 1)
        ohe = (lax.broadcasted_iota(jnp.int32, (_TBL, NUM_EXPERTS), 1)
               == e_cl).astype(jnp.float32)

        def pick(row):
            return jnp.sum(ohe * row, axis=1, keepdims=True)

        tcum_t = pick(tcum)
        tiles_t = pick(tiles)
        poff_t = pick(poff)
        cnt_t = pick(counts.astype(jnp.float32))
        i_t = tqf - (tcum_t - tiles_t)
        s_t = (poff_t + i_t * BLK).astype(jnp.int32)
        valid = jnp.clip((cnt_t - i_t * BLK).astype(jnp.int32), 0, BLK)
        c_t = jnp.minimum(s_t, TOKENS_PAD - BLK)
        lo = s_t - c_t
        hi = lo + valid
        is_pad = tqf >= total
        eids = lax.broadcasted_iota(jnp.int32, (1, NUM_EXPERTS), 1)
        e_last = jnp.max(jnp.where(counts > 0, eids, 0))
        tbl_ref[pl.ds(0 * _TBL, _TBL), :] = jnp.where(is_pad, e_last, e_cl)
        tbl_ref[pl.ds(1 * _TBL, _TBL), :] = jnp.where(is_pad, 0, c_t)
        tbl_ref[pl.ds(2 * _TBL, _TBL), :] = jnp.where(is_pad, 0, lo)
        tbl_ref[pl.ds(3 * _TBL, _TBL), :] = jnp.where(is_pad, 0, hi)


def _router(x, router_w):
    return pl.pallas_call(
        _router_body,
        grid=(TOKENS // _RBT,),
        in_specs=[
            pl.BlockSpec((_RBT, MODEL_DIM), lambda t: (t, 0)),
            pl.BlockSpec((NUM_EXPERTS, MODEL_DIM), lambda t: (0, 0)),
        ],
        out_specs=[
            pl.BlockSpec((TOKENS, 1), lambda t: (0, 0)),
            pl.BlockSpec((4 * _TBL, 1), lambda t: (0, 0)),
        ],
        out_shape=[
            jax.ShapeDtypeStruct((TOKENS, 1), jnp.int32),
            jax.ShapeDtypeStruct((4 * _TBL, 1), jnp.int32),
        ],
        scratch_shapes=[
            pltpu.VMEM((1, NUM_EXPERTS), jnp.int32),
            pltpu.VMEM((TOKENS, 1), jnp.int32),
            pltpu.VMEM((TOKENS, 1), jnp.int32),
        ],
        compiler_params=pltpu.CompilerParams(
            dimension_semantics=("arbitrary",)),
    )(x, router_w)


_SC_CORES = 2
_SC_SUBCORES = 16
_SC_WORKERS = _SC_CORES * _SC_SUBCORES
_SC_CHUNK = 128


def _sc_gather_rows(table, idx):
    n_rows = idx.shape[0]
    assert n_rows % (_SC_WORKERS * PAD8) == 0
    rows_pw = n_rows // _SC_WORKERS
    subs = [_SC_CHUNK] * (rows_pw // _SC_CHUNK)
    if rows_pw % _SC_CHUNK:
        subs.append(rows_pw % _SC_CHUNK)
    mesh = plsc.VectorSubcoreMesh(core_axis_name="c", subcore_axis_name="s")

    @functools.partial(
        pl.kernel,
        mesh=mesh,
        out_type=jax.ShapeDtypeStruct((n_rows, MODEL_DIM), jnp.float32),
        scratch_types=[
            *[pltpu.VMEM((s,), jnp.int32) for s in subs],
            pltpu.VMEM((rows_pw, MODEL_DIM), jnp.float32),
            pltpu.SemaphoreType.DMA,
        ],
    )
    def k(table_hbm, idx_hbm, out_hbm, *refs):
        idx_vs = refs[:len(subs)]
        rows_v, sem = refs[len(subs)], refs[len(subs) + 1]
        wid = lax.axis_index("s") * _SC_CORES + lax.axis_index("c")
        base = wid * rows_pw
        copies = []
        off = 0
        for s, idx_v in zip(subs, idx_vs):
            pltpu.sync_copy(idx_hbm.at[pl.ds(base + off, s)], idx_v)
            copies.append(
                pltpu.async_copy(table_hbm.at[idx_v],
                                 rows_v.at[pl.ds(off, s)], sem))
            off += s
        for c in copies:
            c.wait()
        pltpu.sync_copy(rows_v, out_hbm.at[pl.ds(base, rows_pw)])

    return k(table, idx)


def _sc_scatter_rows(x, slot):
    rows_pw = TOKENS // _SC_WORKERS
    mesh = plsc.VectorSubcoreMesh(core_axis_name="c", subcore_axis_name="s")

    @functools.partial(
        pl.kernel,
        mesh=mesh,
        out_type=jax.ShapeDtypeStruct((TOKENS_PAD, MODEL_DIM), jnp.float32),
        scratch_types=[
            pltpu.VMEM((rows_pw, MODEL_DIM), jnp.float32),
            pltpu.VMEM((rows_pw,), jnp.int32),
            pltpu.SemaphoreType.DMA,
        ],
    )
    def k(x_hbm, slot_hbm, xs_hbm, rows_v, slot_v, sem):
        wid = lax.axis_index("s") * _SC_CORES + lax.axis_index("c")
        base = wid * rows_pw
        cp_rows = pltpu.async_copy(x_hbm.at[pl.ds(base, rows_pw)], rows_v, sem)
        pltpu.sync_copy(slot_hbm.at[pl.ds(base, rows_pw)], slot_v)
        cp_rows.wait()
        pltpu.async_copy(rows_v, xs_hbm.at[slot_v], sem).wait()

    return k(x, slot)


def _gemm_body(e_ref, c_ref, lo_ref, hi_ref,
               xs_ref, w1_ref, w3_ref, w2_ref, out_ref):
    t = pl.program_id(0)
    c = pl.multiple_of(c_ref[t], PAD8)
    lo = lo_ref[t]
    hi = hi_ref[t]

    @pl.when(hi > lo)
    def _():
        xb = xs_ref[pl.ds(c, BLK), :]
        w1e = w1_ref[0]
        w3e = w3_ref[0]
        w2e = w2_ref[0]
        nt = (((1,), (1,)), ((), ()))
        a = lax.dot_general(xb, w1e, nt, preferred_element_type=jnp.float32)
        b = lax.dot_general(xb, w3e, nt, preferred_element_type=jnp.float32)
        h = (a * jax.nn.sigmoid(a)) * b
        ob = lax.dot_general(h, w2e, nt, preferred_element_type=jnp.float32)
        rows = lax.broadcasted_iota(jnp.int32, (BLK, 1), 0)
        keep = (rows >= lo) & (rows < hi)
        cur = out_ref[pl.ds(c, BLK), :]
        out_ref[pl.ds(c, BLK), :] = jnp.where(keep, ob, cur)


def _grouped_gemm(e_arr, c_arr, lo_arr, hi_arr, xs, w1, w3, w2):
    spec = pltpu.PrefetchScalarGridSpec(
        num_scalar_prefetch=4,
        grid=(MAX_TILES,),
        in_specs=[
            pl.BlockSpec((TOKENS_PAD, MODEL_DIM), lambda t, e, c, l, h: (0, 0)),
            pl.BlockSpec((1, EXPERT_DIM, MODEL_DIM),
                         lambda t, e, c, l, h: (e[t], 0, 0)),
            pl.BlockSpec((1, EXPERT_DIM, MODEL_DIM),
                         lambda t, e, c, l, h: (e[t], 0, 0)),
            pl.BlockSpec((1, MODEL_DIM, EXPERT_DIM),
                         lambda t, e, c, l, h: (e[t], 0, 0)),
        ],
        out_specs=pl.BlockSpec((TOKENS_PAD, MODEL_DIM),
                               lambda t, e, c, l, h: (0, 0)),
    )
    return pl.pallas_call(
        _gemm_body,
        grid_spec=spec,
        out_shape=jax.ShapeDtypeStruct((TOKENS_PAD, MODEL_DIM), jnp.float32),
        compiler_params=pltpu.CompilerParams(
            dimension_semantics=("arbitrary",)),
    )(e_arr, c_arr, lo_arr, hi_arr, xs, w1, w3, w2)


def kernel(x, router_w, w1, w2, w3):
    slot4, tbl = _router(x, router_w)
    slot = slot4.reshape(TOKENS)
    tbl = tbl.reshape(4 * _TBL)

    xs = _sc_scatter_rows(x, slot)
    os_ = _grouped_gemm(tbl[:_TBL], tbl[_TBL:2 * _TBL],
                        tbl[2 * _TBL:3 * _TBL], tbl[3 * _TBL:],
                        xs, w1, w3, w2)
    return _sc_gather_rows(os_, slot)
